# baseline (speedup 1.0000x reference)
import math

import jax
import jax.numpy as jnp
import numpy as np
from jax import lax
from jax.experimental import pallas as pl
from jax.experimental.pallas import tpu as pltpu
from jax.experimental.pallas import tpu_sc as plsc

VOCAB = 1000000
SIZE = 64
MAX_SEQ_LEN = 1000
BATCH = 4096
SEQ = 200

NUM_CORES = 2
NUM_SUBCORES = 16
NUM_WORKERS = NUM_CORES * NUM_SUBCORES

ROWS_TOTAL = BATCH * SEQ
ROWS_PER_WORKER = ROWS_TOTAL // NUM_WORKERS
CHUNK = 128
CHUNKS_PER_WORKER = ROWS_PER_WORKER // CHUNK
LANES = 16
VECS_PER_ROW = SIZE // LANES
SCALE = math.sqrt(SIZE)


def _make_pe2():
    pe = np.zeros((MAX_SEQ_LEN, SIZE), dtype=np.float32)
    position = np.arange(0, MAX_SEQ_LEN, dtype=np.float32)[:, None]
    div_term = np.exp(
        np.arange(0, SIZE, 2, dtype=np.float32) * -(math.log(10000.0) / SIZE))
    pe[:, 0::2] = np.sin(position * div_term)
    pe[:, 1::2] = np.cos(position * div_term)
    pe = pe[:SEQ]
    return np.concatenate([pe, pe], axis=0)


_PE2 = _make_pe2()


def _sc_body(x_hbm, table_hbm, pe2_hbm, out_hbm,
             idx_v, pe2_v, rows0, rows1, outb0, outb1,
             gsem0, gsem1, osem0, osem1):
    wid = lax.axis_index("s") * NUM_CORES + lax.axis_index("c")
    base = wid * ROWS_PER_WORKER

    pltpu.sync_copy(x_hbm.at[pl.ds(base, ROWS_PER_WORKER)], idx_v)
    pltpu.sync_copy(pe2_hbm, pe2_v)

    rows_bufs = (rows0, rows1)
    out_bufs = (outb0, outb1)
    gsems = (gsem0, gsem1)
    osems = (osem0, osem1)

    def gather(ci, b):
        return pltpu.make_async_copy(
            table_hbm.at[idx_v.at[pl.ds(ci * CHUNK, CHUNK)]],
            rows_bufs[b], gsems[b])

    def out_copy(ci, b):
        return pltpu.make_async_copy(
            out_bufs[b], out_hbm.at[pl.ds(base + ci * CHUNK, CHUNK)],
            osems[b])

    gather(0, 0).start()
    gather(1, 1).start()

    def step(i, _):
        for b in range(2):
            ci = i * 2 + b
            rows = rows_bufs[b]
            outb = out_bufs[b]
            off = lax.rem(ci * CHUNK, SEQ)
            gather(ci, b).wait()

            @pl.when(ci >= 2)
            def _():
                out_copy(ci - 2, b).wait()

            def row_body(r, _):
                for c in range(VECS_PER_ROW):
                    sl = pl.ds(c * LANES, LANES)
                    outb[r, sl] = rows[r, sl] * SCALE + pe2_v[off + r, sl]
                return 0

            lax.fori_loop(0, CHUNK, row_body, 0, unroll=2)

            @pl.when(ci + 2 < CHUNKS_PER_WORKER)
            def _():
                gather(ci + 2, b).start()

            out_copy(ci, b).start()
        return 0

    lax.fori_loop(0, CHUNKS_PER_WORKER // 2, step, 0)

    out_copy(CHUNKS_PER_WORKER - 2, 0).wait()
    out_copy(CHUNKS_PER_WORKER - 1, 1).wait()


@jax.jit
def kernel(x, emb_table):
    x_flat = x.reshape(-1).astype(jnp.int32)
    pe2 = jnp.asarray(_PE2)
    mesh = plsc.VectorSubcoreMesh(core_axis_name="c", subcore_axis_name="s")
    out = pl.kernel(
        _sc_body,
        out_type=jax.ShapeDtypeStruct((ROWS_TOTAL, SIZE), jnp.float32),
        mesh=mesh,
        compiler_params=pltpu.CompilerParams(use_tc_tiling_on_sc=False),
        scratch_types=[
            pltpu.VMEM((ROWS_PER_WORKER,), jnp.int32),
            pltpu.VMEM((2 * SEQ, SIZE), jnp.float32),
            pltpu.VMEM((CHUNK, SIZE), jnp.float32),
            pltpu.VMEM((CHUNK, SIZE), jnp.float32),
            pltpu.VMEM((CHUNK, SIZE), jnp.float32),
            pltpu.VMEM((CHUNK, SIZE), jnp.float32),
            pltpu.SemaphoreType.DMA,
            pltpu.SemaphoreType.DMA,
            pltpu.SemaphoreType.DMA,
            pltpu.SemaphoreType.DMA,
        ],
    )(x_flat, emb_table, pe2)
    return out.reshape(BATCH, SEQ, SIZE)
